# baseline (speedup 1.0000x reference)
import jax
import jax.numpy as jnp
from jax.experimental import pallas as pl
from jax.experimental.pallas import tpu as pltpu

_CHUNK = 16


def _loss_kernel(wb_ref, tgt_ref, pred_ref, out_ref, m_ref, g_ref, d_ref,
                 sm_ref):
    NB, CP = pred_ref.shape
    H, W, _ = tgt_ref.shape
    P = H * W
    C = CP // P
    chx = min(_CHUNK, W)
    chy = min(_CHUNK, H)

    tgt = tgt_ref[...]
    cls = jax.lax.broadcasted_iota(jnp.int32, (C, H, W, NB), 0)
    m_ref[...] = jnp.where(tgt[None, :, :, :] == cls,
                           jnp.float32(0.0), jnp.float32(1e30))

    xio = jax.lax.broadcasted_iota(
        jnp.int32, (1, 1, W, 1), 2).astype(jnp.float32)

    def s1_chunk(base_f, quad, g):
        for k in range(quad.shape[2]):
            dx = xio - (base_f + float(k))
            upd = quad[:, :, k:k + 1, :] + dx * dx
            g = upd if g is None else jnp.minimum(g, upd)
        return g

    g_ref[...] = s1_chunk(0.0, m_ref[:, :, 0:chx, :], None)

    def s1(i, carry):
        base = i * chx
        quad = m_ref[:, :, pl.ds(base, chx), :]
        g_ref[...] = s1_chunk(base.astype(jnp.float32), quad, g_ref[...])
        return carry

    jax.lax.fori_loop(1, W // chx, s1, 0)

    yio = jax.lax.broadcasted_iota(
        jnp.int32, (1, H, 1, 1), 1).astype(jnp.float32)

    def s2_chunk(base_f, quad, d):
        for k in range(quad.shape[1]):
            dy = yio - (base_f + float(k))
            upd = quad[:, k:k + 1, :, :] + dy * dy
            d = upd if d is None else jnp.minimum(d, upd)
        return d

    d_ref[...] = s2_chunk(0.0, g_ref[:, 0:chy, :, :], None)

    def s2(i, carry):
        base = i * chy
        quad = g_ref[:, pl.ds(base, chy), :, :]
        d_ref[...] = s2_chunk(base.astype(jnp.float32), quad, d_ref[...])
        return carry

    jax.lax.fori_loop(1, H // chy, s2, 0)

    dist = jnp.sqrt(d_ref[...])
    dmax = jnp.max(dist, axis=(1, 2))

    wb = wb_ref[...][:, None, :, :]
    rhs = jnp.where(m_ref[...] == 0.0,
                    (-wb * dmax[:, None, None, :]),
                    wb * dist)
    rhs_bf = rhs.reshape(CP, NB).astype(jnp.bfloat16)

    logits = pred_ref[...]
    mx = logits[:, 0:P]
    for c in range(1, C):
        mx = jnp.maximum(mx, logits[:, c * P:(c + 1) * P])
    den = jnp.zeros((NB, P), jnp.float32)
    for c in range(C):
        ex_c = jnp.exp(logits[:, c * P:(c + 1) * P] - mx)
        sm_ref[:, c * P:(c + 1) * P] = ex_c
        den = den + ex_c
    rden = 1.0 / den
    sm_bf = jnp.concatenate(
        [(sm_ref[:, c * P:(c + 1) * P] * rden).astype(jnp.bfloat16)
         for c in range(C)], axis=1)

    acc = jnp.dot(sm_bf, rhs_bf,
                  preferred_element_type=jnp.float32)
    eye = (jax.lax.broadcasted_iota(jnp.int32, (NB, NB), 0) ==
           jax.lax.broadcasted_iota(jnp.int32, (NB, NB), 1)
           ).astype(jnp.float32)
    total = jnp.sum(acc * eye, axis=0, keepdims=True)
    out_ref[...] = total[None]


def kernel(predictions, targets, weight):
    nb, nc, h, width = predictions.shape
    p = h * width

    if weight is None or len(weight) != nc:
        weight_arr = jnp.ones((nc,), jnp.float32)
    else:
        weight_arr = jnp.asarray(weight, jnp.float32)
    w_norm = (weight_arr / (jnp.sum(weight_arr) * (nb * nc * p))
              ).astype(jnp.float32)

    NB = 128
    num_blocks = nb // NB

    preds_r = predictions.astype(jnp.float32).reshape(nb, nc * p)
    tgts_t = jnp.transpose(targets.astype(jnp.int32), (1, 2, 0))
    w_b = jnp.broadcast_to(w_norm[:, None, None], (nc, 1, NB))

    grid_spec = pltpu.PrefetchScalarGridSpec(
        num_scalar_prefetch=0,
        grid=(num_blocks,),
        in_specs=[
            pl.BlockSpec((nc, 1, NB), lambda i: (0, 0, 0)),
            pl.BlockSpec((h, width, NB), lambda i: (0, 0, i)),
            pl.BlockSpec((NB, nc * p), lambda i: (i, 0)),
        ],
        out_specs=pl.BlockSpec((1, 1, NB), lambda i: (i, 0, 0)),
        scratch_shapes=[
            pltpu.VMEM((nc, h, width, NB), jnp.float32),
            pltpu.VMEM((nc, h, width, NB), jnp.float32),
            pltpu.VMEM((nc, h, width, NB), jnp.float32),
            pltpu.VMEM((NB, nc * p), jnp.float32),
        ],
    )

    partials = pl.pallas_call(
        _loss_kernel,
        out_shape=jax.ShapeDtypeStruct((num_blocks, 1, NB), jnp.float32),
        grid_spec=grid_spec,
        compiler_params=pltpu.CompilerParams(
            dimension_semantics=("parallel",),
            vmem_limit_bytes=64 * 1024 * 1024),
    )(w_b, tgts_t, preds_r)

    return jnp.sum(partials)


# speedup vs baseline: 1.3992x; 1.3992x over previous
import jax
import jax.numpy as jnp
from jax.experimental import pallas as pl
from jax.experimental.pallas import tpu as pltpu

_CHUNK = 8


def _loss_kernel(wb_ref, tgt_ref, pred_ref, out_ref, m_ref, g_ref, d_ref,
                 sm_ref):
    NB, CP = pred_ref.shape
    H, W, _ = tgt_ref.shape
    P = H * W
    C = CP // P

    tgt = tgt_ref[...]
    cls = jax.lax.broadcasted_iota(jnp.int32, (C, H, W, NB), 0)
    m_ref[...] = jnp.where(tgt[None, :, :, :] == cls,
                           jnp.float32(0.0), jnp.float32(1e30))

    xio = jax.lax.broadcasted_iota(
        jnp.int32, (1, 1, W, 1), 2).astype(jnp.float32)

    def s1_chunk(base_f, quad, g):
        for k in range(_CHUNK):
            dx = xio - (base_f + float(k))
            upd = quad[:, :, k:k + 1, :] + dx * dx
            g = upd if g is None else jnp.minimum(g, upd)
        return g

    g_ref[...] = s1_chunk(0.0, m_ref[:, :, 0:_CHUNK, :], None)

    def s1(i, carry):
        base = i * _CHUNK
        quad = m_ref[:, :, pl.ds(base, _CHUNK), :]
        g_ref[...] = s1_chunk(base.astype(jnp.float32), quad, g_ref[...])
        return carry

    jax.lax.fori_loop(1, W // _CHUNK, s1, 0)

    yio = jax.lax.broadcasted_iota(
        jnp.int32, (1, H, 1, 1), 1).astype(jnp.float32)

    def s2_chunk(base_f, quad, d):
        for k in range(_CHUNK):
            dy = yio - (base_f + float(k))
            upd = quad[:, k:k + 1, :, :] + dy * dy
            d = upd if d is None else jnp.minimum(d, upd)
        return d

    d_ref[...] = s2_chunk(0.0, g_ref[:, 0:_CHUNK, :, :], None)

    def s2(i, carry):
        base = i * _CHUNK
        quad = g_ref[:, pl.ds(base, _CHUNK), :, :]
        d_ref[...] = s2_chunk(base.astype(jnp.float32), quad, d_ref[...])
        return carry

    jax.lax.fori_loop(1, H // _CHUNK, s2, 0)

    dist = jnp.sqrt(d_ref[...])
    dmax = jnp.max(dist, axis=(1, 2))

    wb = wb_ref[...][:, None, :, :]
    rhs = jnp.where(m_ref[...] == 0.0,
                    (-wb * dmax[:, None, None, :]),
                    wb * dist)
    rhs_bf = rhs.reshape(CP, NB).astype(jnp.bfloat16)

    logits = pred_ref[...]
    mx = logits[:, 0:P]
    for c in range(1, C):
        mx = jnp.maximum(mx, logits[:, c * P:(c + 1) * P])
    den = jnp.zeros((NB, P), jnp.float32)
    for c in range(C):
        ex_c = jnp.exp(logits[:, c * P:(c + 1) * P] - mx)
        sm_ref[:, c * P:(c + 1) * P] = ex_c
        den = den + ex_c
    rden = 1.0 / den
    sm_bf = jnp.concatenate(
        [(sm_ref[:, c * P:(c + 1) * P] * rden).astype(jnp.bfloat16)
         for c in range(C)], axis=1)

    acc = jnp.dot(sm_bf, rhs_bf,
                  preferred_element_type=jnp.float32)
    eye = (jax.lax.broadcasted_iota(jnp.int32, (NB, NB), 0) ==
           jax.lax.broadcasted_iota(jnp.int32, (NB, NB), 1)
           ).astype(jnp.float32)
    total = jnp.sum(acc * eye, axis=0, keepdims=True)
    out_ref[...] = total[None]


def kernel(predictions, targets, weight):
    nb, nc, h, width = predictions.shape
    p = h * width

    if weight is None or len(weight) != nc:
        weight_arr = jnp.ones((nc,), jnp.float32)
    else:
        weight_arr = jnp.asarray(weight, jnp.float32)
    w_norm = (weight_arr / (jnp.sum(weight_arr) * (nb * nc * p))
              ).astype(jnp.float32)

    NB = 128
    num_blocks = nb // NB

    preds_r = predictions.astype(jnp.float32).reshape(nb, nc * p)
    tgts_t = jnp.transpose(targets.astype(jnp.int32), (1, 2, 0))
    w_b = jnp.broadcast_to(w_norm[:, None, None], (nc, 1, NB))

    grid_spec = pltpu.PrefetchScalarGridSpec(
        num_scalar_prefetch=0,
        grid=(num_blocks,),
        in_specs=[
            pl.BlockSpec((nc, 1, NB), lambda i: (0, 0, 0)),
            pl.BlockSpec((h, width, NB), lambda i: (0, 0, i)),
            pl.BlockSpec((NB, nc * p), lambda i: (i, 0)),
        ],
        out_specs=pl.BlockSpec((1, 1, NB), lambda i: (i, 0, 0)),
        scratch_shapes=[
            pltpu.VMEM((nc, h, width, NB), jnp.float32),
            pltpu.VMEM((nc, h, width, NB), jnp.float32),
            pltpu.VMEM((nc, h, width, NB), jnp.float32),
            pltpu.VMEM((NB, nc * p), jnp.float32),
        ],
    )

    partials = pl.pallas_call(
        _loss_kernel,
        out_shape=jax.ShapeDtypeStruct((num_blocks, 1, NB), jnp.float32),
        grid_spec=grid_spec,
        compiler_params=pltpu.CompilerParams(
            dimension_semantics=("parallel",),
            vmem_limit_bytes=64 * 1024 * 1024),
    )(w_b, tgts_t, preds_r)

    return jnp.sum(partials)
